# baseline (speedup 1.0000x reference)
import functools

import jax
import jax.numpy as jnp
from jax.experimental import pallas as pl
from jax.experimental.pallas import tpu as pltpu

_MIB = 1024 * 1024


def _prep_kernel(wo_ref, wv_ref, wf2_ref):
    wf2_ref[...] = jnp.dot(wo_ref[...].astype(jnp.bfloat16),
                           wv_ref[...].astype(jnp.bfloat16),
                           preferred_element_type=jnp.float32
                           ).astype(jnp.bfloat16)


def _fused_kernel(q_ref, k_ref, v_ref, wq_ref, wk_ref, wf2_ref,
                  out_ref, attn_ref, *, lo, head_dim, scale, inv_heads, nb):
    tb = (((1,), (1,)), ((), ()))
    nL, D = q_ref.shape[0] * q_ref.shape[1], q_ref.shape[2]
    L = q_ref.shape[1]
    v = v_ref[...].astype(jnp.bfloat16).reshape(nL, D)
    out = jax.lax.dot_general(v, wf2_ref[...], tb,
                              preferred_element_type=jnp.float32)
    out_ref[...] = out.reshape(nb, L, out.shape[-1])

    wqh = wq_ref[...].astype(jnp.bfloat16) * jnp.bfloat16(scale)
    wkh = wk_ref[...].astype(jnp.bfloat16)
    q = q_ref[...].astype(jnp.bfloat16).reshape(nL, D)
    k = k_ref[...].astype(jnp.bfloat16).reshape(nL, D)
    qh = jax.lax.dot_general(q, wqh, tb, preferred_element_type=jnp.float32)
    kh = jax.lax.dot_general(k, wkh, tb, preferred_element_type=jnp.float32)
    qh = qh.astype(jnp.bfloat16).reshape(nb, L, head_dim)
    kh = kh.astype(jnp.bfloat16).reshape(nb, L, head_dim)
    for j in range(nb):
        s = jax.lax.dot_general(qh[j], kh[j], tb,
                                preferred_element_type=jnp.float32)
        s = s - jnp.max(s, axis=-1, keepdims=True)
        e = jnp.exp(s)
        attn_ref[j] = e * (inv_heads / jnp.sum(e, axis=-1, keepdims=True))


def kernel(key, value, query, wq, wk, wv, wo):
    num_heads = 8
    B, Lk, Dk = key.shape
    _, Lv, Dv = value.shape
    _, Lq, _ = query.shape
    Dout = wo.shape[0]
    head_dim = Dk // num_heads
    lo = (num_heads - 1) * head_dim
    scale = head_dim ** (-0.5)

    wf2 = pl.pallas_call(
        _prep_kernel,
        out_shape=jax.ShapeDtypeStruct((Dout, Dv), jnp.bfloat16),
        compiler_params=pltpu.CompilerParams(
            vmem_limit_bytes=32 * _MIB),
    )(wo, wv)

    nb = 4 if B % 4 == 0 else 1
    grid_b = B // nb

    wq_h = jax.lax.slice(wq, (lo, 0), (lo + head_dim, Dk))
    wk_h = jax.lax.slice(wk, (lo, 0), (lo + head_dim, Dk))

    kern = functools.partial(_fused_kernel, lo=lo, head_dim=head_dim,
                             scale=scale, inv_heads=1.0 / num_heads, nb=nb)

    in_bytes = nb * 4 * (Lq * Dk + Lk * Dk + Lv * Dv)
    out_bytes = nb * 4 * (Lv * Dout + Lq * Lk)
    w_bytes = 4 * 2 * head_dim * Dk + 2 * Dout * Dv
    vmem = 2 * (in_bytes + out_bytes) + w_bytes + 8 * nb * Lq * Lk * 4

    cost = pl.CostEstimate(
        flops=2 * B * (Lv * Dv * Dout + (Lq + Lk) * Dk * 128 + Lq * Lk * 128),
        transcendentals=B * Lq * Lk,
        bytes_accessed=grid_b * (in_bytes + out_bytes) + w_bytes)

    out, attn = pl.pallas_call(
        kern,
        out_shape=(jax.ShapeDtypeStruct((B, Lv, Dout), jnp.float32),
                   jax.ShapeDtypeStruct((B, Lq, Lk), jnp.float32)),
        grid=(grid_b,),
        in_specs=[
            pl.BlockSpec((nb, Lq, Dk), lambda b: (b, 0, 0)),
            pl.BlockSpec((nb, Lk, Dk), lambda b: (b, 0, 0)),
            pl.BlockSpec((nb, Lv, Dv), lambda b: (b, 0, 0)),
            pl.BlockSpec((head_dim, Dk), lambda b: (0, 0)),
            pl.BlockSpec((head_dim, Dk), lambda b: (0, 0)),
            pl.BlockSpec((Dout, Dv), lambda b: (0, 0)),
        ],
        out_specs=(pl.BlockSpec((nb, Lv, Dout), lambda b: (b, 0, 0)),
                   pl.BlockSpec((nb, Lq, Lk), lambda b: (b, 0, 0))),
        compiler_params=pltpu.CompilerParams(
            dimension_semantics=("parallel",),
            vmem_limit_bytes=int(min(max(vmem, 32 * _MIB), 64 * _MIB))),
        cost_estimate=cost,
    )(query, key, value, wq_h, wk_h, wf2)
    return out, attn


# speedup vs baseline: 1.0410x; 1.0410x over previous
import functools

import jax
import jax.numpy as jnp
from jax.experimental import pallas as pl
from jax.experimental.pallas import tpu as pltpu

_MIB = 1024 * 1024


def _prep_kernel(wo_ref, wv_ref, wf2_ref):
    wf2_ref[...] = jnp.dot(wo_ref[...].astype(jnp.bfloat16),
                           wv_ref[...].astype(jnp.bfloat16),
                           preferred_element_type=jnp.float32
                           ).astype(jnp.bfloat16)


def _fused_kernel(q_ref, k_ref, v_ref, wq_ref, wk_ref, wf2_ref,
                  out_ref, attn_ref, *, lo, head_dim, scale, inv_heads, nb):
    tb = (((1,), (1,)), ((), ()))
    nL, D = q_ref.shape[0] * q_ref.shape[1], q_ref.shape[2]
    L = q_ref.shape[1]
    v = v_ref[...].astype(jnp.bfloat16).reshape(nL, D)
    out = jax.lax.dot_general(v, wf2_ref[...], tb,
                              preferred_element_type=jnp.float32)
    out_ref[...] = out.reshape(nb, L, out.shape[-1])

    wqh = (wq_ref[lo:lo + head_dim, :].astype(jnp.bfloat16)
           * jnp.bfloat16(scale))
    wkh = wk_ref[lo:lo + head_dim, :].astype(jnp.bfloat16)
    q = q_ref[...].astype(jnp.bfloat16).reshape(nL, D)
    k = k_ref[...].astype(jnp.bfloat16).reshape(nL, D)
    qh = jax.lax.dot_general(q, wqh, tb, preferred_element_type=jnp.float32)
    kh = jax.lax.dot_general(k, wkh, tb, preferred_element_type=jnp.float32)
    qh = qh.astype(jnp.bfloat16).reshape(nb, L, head_dim)
    kh = kh.astype(jnp.bfloat16).reshape(nb, L, head_dim)
    for j in range(nb):
        s = jax.lax.dot_general(qh[j], kh[j], tb,
                                preferred_element_type=jnp.float32)
        s = s - jnp.max(s, axis=-1, keepdims=True)
        e = jnp.exp(s)
        attn_ref[j] = e * (inv_heads / jnp.sum(e, axis=-1, keepdims=True))


def kernel(key, value, query, wq, wk, wv, wo):
    num_heads = 8
    B, Lk, Dk = key.shape
    _, Lv, Dv = value.shape
    _, Lq, _ = query.shape
    Dout = wo.shape[0]
    head_dim = Dk // num_heads
    lo = (num_heads - 1) * head_dim
    scale = head_dim ** (-0.5)

    wf2 = pl.pallas_call(
        _prep_kernel,
        out_shape=jax.ShapeDtypeStruct((Dout, Dv), jnp.bfloat16),
        compiler_params=pltpu.CompilerParams(
            vmem_limit_bytes=32 * _MIB),
    )(wo, wv)

    nb = 4 if B % 4 == 0 else 1
    grid_b = B // nb

    kern = functools.partial(_fused_kernel, lo=lo, head_dim=head_dim,
                             scale=scale, inv_heads=1.0 / num_heads, nb=nb)

    in_bytes = nb * 4 * (Lq * Dk + Lk * Dk + Lv * Dv)
    out_bytes = nb * 4 * (Lv * Dout + Lq * Lk)
    w_bytes = 4 * 2 * Dk * Dk + 2 * Dout * Dv
    vmem = 2 * (in_bytes + out_bytes) + w_bytes + 8 * nb * Lq * Lk * 4

    cost = pl.CostEstimate(
        flops=2 * B * (Lv * Dv * Dout + (Lq + Lk) * Dk * 128 + Lq * Lk * 128),
        transcendentals=B * Lq * Lk,
        bytes_accessed=grid_b * (in_bytes + out_bytes) + w_bytes)

    out, attn = pl.pallas_call(
        kern,
        out_shape=(jax.ShapeDtypeStruct((B, Lv, Dout), jnp.float32),
                   jax.ShapeDtypeStruct((B, Lq, Lk), jnp.float32)),
        grid=(grid_b,),
        in_specs=[
            pl.BlockSpec((nb, Lq, Dk), lambda b: (b, 0, 0)),
            pl.BlockSpec((nb, Lk, Dk), lambda b: (b, 0, 0)),
            pl.BlockSpec((nb, Lv, Dv), lambda b: (b, 0, 0)),
            pl.BlockSpec((Dk, Dk), lambda b: (0, 0)),
            pl.BlockSpec((Dk, Dk), lambda b: (0, 0)),
            pl.BlockSpec((Dout, Dv), lambda b: (0, 0)),
        ],
        out_specs=(pl.BlockSpec((nb, Lv, Dout), lambda b: (b, 0, 0)),
                   pl.BlockSpec((nb, Lq, Lk), lambda b: (b, 0, 0))),
        compiler_params=pltpu.CompilerParams(
            dimension_semantics=("arbitrary",),
            vmem_limit_bytes=int(min(max(vmem, 32 * _MIB), 64 * _MIB))),
        cost_estimate=cost,
    )(query, key, value, wq, wk, wf2)
    return out, attn


# speedup vs baseline: 1.0420x; 1.0009x over previous
import functools

import jax
import jax.numpy as jnp
from jax.experimental import pallas as pl
from jax.experimental.pallas import tpu as pltpu

_MIB = 1024 * 1024


def _prep_kernel(wo_ref, wv_ref, wf2_ref):
    wf2_ref[...] = jnp.dot(wo_ref[...].astype(jnp.bfloat16),
                           wv_ref[...].astype(jnp.bfloat16),
                           preferred_element_type=jnp.float32
                           ).astype(jnp.bfloat16)


def _fused_kernel(q_ref, k_ref, v_ref, wq_ref, wk_ref, wf2_ref,
                  out_ref, attn_ref, *, lo, head_dim, scale, inv_heads, nb):
    tb = (((1,), (1,)), ((), ()))
    nL, D = q_ref.shape[0] * q_ref.shape[1], q_ref.shape[2]
    L = q_ref.shape[1]
    v = v_ref[...].astype(jnp.bfloat16).reshape(nL, D)
    out = jax.lax.dot_general(v, wf2_ref[...], tb,
                              preferred_element_type=jnp.float32)
    out_ref[...] = out.reshape(nb, L, out.shape[-1])

    wqh = wq_ref[lo:lo + head_dim, :] * scale
    wkh = wk_ref[lo:lo + head_dim, :]
    q = q_ref[...].reshape(nL, D)
    k = k_ref[...].reshape(nL, D)
    qh = jax.lax.dot_general(q, wqh, tb, preferred_element_type=jnp.float32)
    kh = jax.lax.dot_general(k, wkh, tb, preferred_element_type=jnp.float32)
    qh = qh.reshape(nb, L, head_dim)
    kh = kh.reshape(nb, L, head_dim)
    for j in range(nb):
        s = jax.lax.dot_general(qh[j], kh[j], tb,
                                preferred_element_type=jnp.float32)
        s = s - jnp.max(s, axis=-1, keepdims=True)
        e = jnp.exp(s)
        attn_ref[j] = e * (inv_heads / jnp.sum(e, axis=-1, keepdims=True))


def kernel(key, value, query, wq, wk, wv, wo):
    num_heads = 8
    B, Lk, Dk = key.shape
    _, Lv, Dv = value.shape
    _, Lq, _ = query.shape
    Dout = wo.shape[0]
    head_dim = Dk // num_heads
    lo = (num_heads - 1) * head_dim
    scale = head_dim ** (-0.5)

    wf2 = pl.pallas_call(
        _prep_kernel,
        out_shape=jax.ShapeDtypeStruct((Dout, Dv), jnp.bfloat16),
        compiler_params=pltpu.CompilerParams(
            vmem_limit_bytes=32 * _MIB),
    )(wo, wv)

    nb = 4 if B % 4 == 0 else 1
    grid_b = B // nb

    kern = functools.partial(_fused_kernel, lo=lo, head_dim=head_dim,
                             scale=scale, inv_heads=1.0 / num_heads, nb=nb)

    in_bytes = nb * 4 * (Lq * Dk + Lk * Dk + Lv * Dv)
    out_bytes = nb * 4 * (Lv * Dout + Lq * Lk)
    w_bytes = 4 * 2 * Dk * Dk + 2 * Dout * Dv
    vmem = 2 * (in_bytes + out_bytes) + w_bytes + 8 * nb * Lq * Lk * 4

    cost = pl.CostEstimate(
        flops=2 * B * (Lv * Dv * Dout + (Lq + Lk) * Dk * 128 + Lq * Lk * 128),
        transcendentals=B * Lq * Lk,
        bytes_accessed=grid_b * (in_bytes + out_bytes) + w_bytes)

    out, attn = pl.pallas_call(
        kern,
        out_shape=(jax.ShapeDtypeStruct((B, Lv, Dout), jnp.float32),
                   jax.ShapeDtypeStruct((B, Lq, Lk), jnp.float32)),
        grid=(grid_b,),
        in_specs=[
            pl.BlockSpec((nb, Lq, Dk), lambda b: (b, 0, 0)),
            pl.BlockSpec((nb, Lk, Dk), lambda b: (b, 0, 0)),
            pl.BlockSpec((nb, Lv, Dv), lambda b: (b, 0, 0)),
            pl.BlockSpec((Dk, Dk), lambda b: (0, 0)),
            pl.BlockSpec((Dk, Dk), lambda b: (0, 0)),
            pl.BlockSpec((Dout, Dv), lambda b: (0, 0)),
        ],
        out_specs=(pl.BlockSpec((nb, Lv, Dout), lambda b: (b, 0, 0)),
                   pl.BlockSpec((nb, Lq, Lk), lambda b: (b, 0, 0))),
        compiler_params=pltpu.CompilerParams(
            dimension_semantics=("arbitrary",),
            vmem_limit_bytes=int(min(max(vmem, 32 * _MIB), 64 * _MIB))),
        cost_estimate=cost,
    )(query, key, value, wq, wk, wf2)
    return out, attn
